# Initial kernel scaffold; baseline (speedup 1.0000x reference)
import jax
import jax.numpy as jnp
from jax.experimental import pallas as pl


def kernel(values, lengths, table):
    raise NotImplementedError("write your pallas kernel here")


# speedup vs baseline: 1.4023x; 1.4023x over previous
import functools

import jax
import jax.numpy as jnp
from jax import lax
from jax.experimental import pallas as pl
from jax.experimental.pallas import tpu as pltpu
from jax.experimental.pallas import tpu_sc as plsc

ZCH = 1000000
EMBED_DIM = 64
TOTAL = 425984
NW = 32
CHUNK = TOTAL // NW
ROWS = 128
NJ = CHUNK // ROWS

ZCHUNK = 15616
PER_TILE = 4 * ZCHUNK
TAIL_OFF = 16 * PER_TILE
TAIL = ZCH - TAIL_OFF


def _sc_body(vals_hbm, table_hbm, emb_hbm, rem_hbm, parts_hbm,
             idx_v, rows_v, ones_v, zbuf_v, counts_sp, sem):
    cid = lax.axis_index("c")
    sid = lax.axis_index("s")
    wid = sid * 2 + cid

    def _fill_ones(c, _):
        ones_v[pl.ds(c * 16, 16)] = jnp.ones((16,), jnp.float32)
        return _
    lax.fori_loop(0, ROWS // 16, _fill_ones, None)

    def _fill_zeros(c, _):
        zbuf_v[pl.ds(c * 16, 16)] = jnp.zeros((16,), jnp.float32)
        return _
    lax.fori_loop(0, ZCHUNK // 16, _fill_zeros, None)

    for q in range(4):
        pltpu.sync_copy(zbuf_v,
                        counts_sp.at[pl.ds(sid * PER_TILE + q * ZCHUNK, ZCHUNK)])

    @pl.when(sid == 15)
    def _zero_tail():
        pltpu.sync_copy(zbuf_v.at[pl.ds(0, TAIL)],
                        counts_sp.at[pl.ds(TAIL_OFF, TAIL)])

    pltpu.sync_copy(vals_hbm.at[wid], idx_v)

    def _mod_row(j, _):
        for c in range(ROWS // 16):
            v = idx_v[j, pl.ds(c * 16, 16)]
            idx_v[j, pl.ds(c * 16, 16)] = lax.rem(v, ZCH)
        return _
    lax.fori_loop(0, NJ, _mod_row, None)

    pltpu.sync_copy(idx_v, rem_hbm.at[wid])

    plsc.subcore_barrier()

    def _gather_row(j, _):
        pltpu.async_copy(table_hbm.at[idx_v.at[j]], rows_v, sem).wait()
        pltpu.sync_copy(rows_v, emb_hbm.at[wid, j])
        pltpu.sync_copy(ones_v, counts_sp.at[idx_v.at[j]], add=True)
        return _
    lax.fori_loop(0, NJ, _gather_row, None)

    plsc.subcore_barrier()

    off = sid * PER_TILE
    pltpu.sync_copy(counts_sp.at[pl.ds(off, PER_TILE)],
                    parts_hbm.at[cid, pl.ds(off, PER_TILE)])

    @pl.when(sid == 15)
    def _dump_tail():
        pltpu.sync_copy(counts_sp.at[pl.ds(TAIL_OFF, TAIL)],
                        parts_hbm.at[cid, pl.ds(TAIL_OFF, TAIL)])


_sc_kernel = functools.partial(
    pl.kernel,
    out_type=[
        jax.ShapeDtypeStruct((NW, NJ, ROWS, EMBED_DIM), jnp.float32),
        jax.ShapeDtypeStruct((NW, NJ, ROWS), jnp.int32),
        jax.ShapeDtypeStruct((2, ZCH), jnp.float32),
    ],
    mesh=plsc.VectorSubcoreMesh(core_axis_name="c", subcore_axis_name="s"),
    compiler_params=pltpu.CompilerParams(use_tc_tiling_on_sc=False),
    scratch_types=[
        pltpu.VMEM((NJ, ROWS), jnp.int32),
        pltpu.VMEM((ROWS, EMBED_DIM), jnp.float32),
        pltpu.VMEM((ROWS,), jnp.float32),
        pltpu.VMEM((ZCHUNK,), jnp.float32),
        pltpu.VMEM_SHARED((ZCH,), jnp.float32),
        pltpu.SemaphoreType.DMA,
    ],
)(_sc_body)


def _combine_body(p_ref, o_ref):
    o_ref[...] = p_ref[0] + p_ref[1]


def kernel(values, lengths, table):
    del lengths
    vals = values.reshape(NW, NJ, ROWS)
    emb, remapped, parts = _sc_kernel(vals, table)
    counts = pl.pallas_call(
        _combine_body,
        out_shape=jax.ShapeDtypeStruct((1000, 1000), jnp.float32),
    )(parts.reshape(2, 1000, 1000))
    return (emb.reshape(TOTAL, EMBED_DIM),
            remapped.reshape(TOTAL),
            counts.reshape(ZCH))


# speedup vs baseline: 1.4861x; 1.0598x over previous
import functools

import jax
import jax.numpy as jnp
from jax import lax
from jax.experimental import pallas as pl
from jax.experimental.pallas import tpu as pltpu
from jax.experimental.pallas import tpu_sc as plsc

ZCH = 1000000
EMBED_DIM = 64
TOTAL = 425984
NW = 32
CHUNK = TOTAL // NW
ROWS = 128
NJ = CHUNK // ROWS

ZCHUNK = 15616
PER_TILE = 4 * ZCHUNK
TAIL_OFF = 16 * PER_TILE
TAIL = ZCH - TAIL_OFF


GK = 2
NG = NJ // GK


def _sc_body(vals_hbm, table_hbm, emb_hbm, rem_hbm, parts_hbm,
             idx_v, buf_a, buf_b, ones_v, zbuf_v, counts_sp,
             gsem_a, gsem_b, wsem_a, wsem_b, csem):
    cid = lax.axis_index("c")
    sid = lax.axis_index("s")
    wid = sid * 2 + cid

    def _fill_ones(c, _):
        ones_v[pl.ds(c * 16, 16)] = jnp.ones((16,), jnp.float32)
        return _
    lax.fori_loop(0, ROWS // 16, _fill_ones, None)

    def _fill_zeros(c, _):
        zbuf_v[pl.ds(c * 16, 16)] = jnp.zeros((16,), jnp.float32)
        return _
    lax.fori_loop(0, ZCHUNK // 16, _fill_zeros, None)

    for q in range(4):
        pltpu.sync_copy(zbuf_v,
                        counts_sp.at[pl.ds(sid * PER_TILE + q * ZCHUNK, ZCHUNK)])

    @pl.when(sid == 15)
    def _zero_tail():
        pltpu.sync_copy(zbuf_v.at[pl.ds(0, TAIL)],
                        counts_sp.at[pl.ds(TAIL_OFF, TAIL)])

    pltpu.sync_copy(vals_hbm.at[wid], idx_v)

    def _mod_row(j, _):
        for c in range(ROWS // 16):
            v = idx_v[j, pl.ds(c * 16, 16)]
            idx_v[j, pl.ds(c * 16, 16)] = lax.rem(v, ZCH)
        return _
    lax.fori_loop(0, NJ, _mod_row, None)

    pltpu.sync_copy(idx_v, rem_hbm.at[wid])

    plsc.subcore_barrier()

    bufs = (buf_a, buf_b)
    gsems = (gsem_a, gsem_b)
    wsems = (wsem_a, wsem_b)

    def _fire_group(g, p):
        return [pltpu.async_copy(table_hbm.at[idx_v.at[g * GK + i]],
                                 bufs[p].at[i], gsems[p])
                for i in range(GK)]

    gpend = {0: _fire_group(0, 0), 1: None}
    wpend = {0: None, 1: None}
    cpend = {0: [], 1: []}
    for g in range(NG):
        p = g % 2
        for c in gpend[p]:
            c.wait()
        if g + 1 < NG:
            if wpend[1 - p] is not None:
                wpend[1 - p].wait()
            gpend[1 - p] = _fire_group(g + 1, 1 - p)
        wpend[p] = pltpu.async_copy(bufs[p], emb_hbm.at[wid, pl.ds(g * GK, GK)],
                                    wsems[p])
        for c in cpend[p]:
            c.wait()
        cpend[p] = [pltpu.async_copy(ones_v, counts_sp.at[idx_v.at[g * GK + i]],
                                     csem, add=True)
                    for i in range(GK)]
    for p in range(2):
        if wpend[p] is not None:
            wpend[p].wait()
        for c in cpend[p]:
            c.wait()

    plsc.subcore_barrier()

    off = sid * PER_TILE
    pltpu.sync_copy(counts_sp.at[pl.ds(off, PER_TILE)],
                    parts_hbm.at[cid, pl.ds(off, PER_TILE)])

    @pl.when(sid == 15)
    def _dump_tail():
        pltpu.sync_copy(counts_sp.at[pl.ds(TAIL_OFF, TAIL)],
                        parts_hbm.at[cid, pl.ds(TAIL_OFF, TAIL)])


_sc_kernel = functools.partial(
    pl.kernel,
    out_type=[
        jax.ShapeDtypeStruct((NW, NJ, ROWS, EMBED_DIM), jnp.float32),
        jax.ShapeDtypeStruct((NW, NJ, ROWS), jnp.int32),
        jax.ShapeDtypeStruct((2, ZCH), jnp.float32),
    ],
    mesh=plsc.VectorSubcoreMesh(core_axis_name="c", subcore_axis_name="s"),
    compiler_params=pltpu.CompilerParams(use_tc_tiling_on_sc=False),
    scratch_types=[
        pltpu.VMEM((NJ, ROWS), jnp.int32),
        pltpu.VMEM((GK, ROWS, EMBED_DIM), jnp.float32),
        pltpu.VMEM((GK, ROWS, EMBED_DIM), jnp.float32),
        pltpu.VMEM((ROWS,), jnp.float32),
        pltpu.VMEM((ZCHUNK,), jnp.float32),
        pltpu.VMEM_SHARED((ZCH,), jnp.float32),
        pltpu.SemaphoreType.DMA,
        pltpu.SemaphoreType.DMA,
        pltpu.SemaphoreType.DMA,
        pltpu.SemaphoreType.DMA,
        pltpu.SemaphoreType.DMA,
    ],
)(_sc_body)


def _combine_body(p_ref, o_ref):
    o_ref[...] = p_ref[0] + p_ref[1]


def kernel(values, lengths, table):
    del lengths
    vals = values.reshape(NW, NJ, ROWS)
    emb, remapped, parts = _sc_kernel(vals, table)
    counts = pl.pallas_call(
        _combine_body,
        out_shape=jax.ShapeDtypeStruct((1000, 1000), jnp.float32),
    )(parts.reshape(2, 1000, 1000))
    return (emb.reshape(TOTAL, EMBED_DIM),
            remapped.reshape(TOTAL),
            counts.reshape(ZCH))


# speedup vs baseline: 1.4998x; 1.0092x over previous
import functools

import jax
import jax.numpy as jnp
from jax import lax
from jax.experimental import pallas as pl
from jax.experimental.pallas import tpu as pltpu
from jax.experimental.pallas import tpu_sc as plsc

ZCH = 1000000
EMBED_DIM = 64
TOTAL = 425984
NW = 32
CHUNK = TOTAL // NW
ROWS = 128
NJ = CHUNK // ROWS
GK = 2
NG = NJ // GK

ZCHUNK = 3904
PER_TILE = 16 * ZCHUNK
TAIL_OFF = 16 * PER_TILE
TAIL = ZCH - TAIL_OFF


def _sc_body(vals_hbm, table_hbm, emb_hbm, rem_hbm, part0_hbm, part1_hbm,
             vals_v, idx_v, buf_a, buf_b, ones_v, zbuf_v, counts_sp,
             gsem_a, gsem_b, wsem_a, wsem_b, csem):
    cid = lax.axis_index("c")
    sid = lax.axis_index("s")
    wid = sid * 2 + cid
    base = wid * CHUNK

    def _fill_ones(c, _):
        ones_v[pl.ds(c * 16, 16)] = jnp.ones((16,), jnp.float32)
        return _
    lax.fori_loop(0, ROWS // 16, _fill_ones, None)

    def _fill_zeros(c, _):
        zbuf_v[pl.ds(c * 16, 16)] = jnp.zeros((16,), jnp.float32)
        return _
    lax.fori_loop(0, ZCHUNK // 16, _fill_zeros, None)

    for q in range(16):
        pltpu.sync_copy(zbuf_v,
                        counts_sp.at[pl.ds(sid * PER_TILE + q * ZCHUNK, ZCHUNK)])

    @pl.when(sid == 15)
    def _zero_tail():
        pltpu.sync_copy(zbuf_v.at[pl.ds(0, TAIL)],
                        counts_sp.at[pl.ds(TAIL_OFF, TAIL)])

    pltpu.sync_copy(vals_hbm.at[pl.ds(base, CHUNK)], vals_v)

    def _mod_row(j, _):
        for c in range(ROWS // 16):
            v = vals_v[pl.ds(j * ROWS + c * 16, 16)]
            r = lax.rem(v, ZCH)
            vals_v[pl.ds(j * ROWS + c * 16, 16)] = r
            idx_v[j, pl.ds(c * 16, 16)] = r
        return _
    lax.fori_loop(0, NJ, _mod_row, None)

    rem_cp = pltpu.async_copy(vals_v, rem_hbm.at[pl.ds(base, CHUNK)], wsem_a)

    plsc.subcore_barrier()

    bufs = (buf_a, buf_b)
    gsems = (gsem_a, gsem_b)
    wsems = (wsem_a, wsem_b)
    parts = (part0_hbm, part1_hbm)

    def _fire_group(g, p):
        return [pltpu.async_copy(table_hbm.at[idx_v.at[g * GK + i]],
                                 bufs[p].at[i], gsems[p])
                for i in range(GK)]

    gpend = {0: _fire_group(0, 0), 1: None}
    wpend = {0: [rem_cp], 1: []}
    cpend = {0: [], 1: []}
    for g in range(NG):
        p = g % 2
        for c in gpend[p]:
            c.wait()
        if g + 1 < NG:
            for c in wpend[1 - p]:
                c.wait()
            gpend[1 - p] = _fire_group(g + 1, 1 - p)
        wpend[p] = [
            pltpu.async_copy(
                bufs[p].at[i],
                emb_hbm.at[pl.ds(base + (g * GK + i) * ROWS, ROWS)],
                wsems[p])
            for i in range(GK)]
        for c in cpend[p]:
            c.wait()
        cpend[p] = [pltpu.async_copy(ones_v, counts_sp.at[idx_v.at[g * GK + i]],
                                     csem, add=True)
                    for i in range(GK)]
    for p in range(2):
        for c in wpend[p]:
            c.wait()
        for c in cpend[p]:
            c.wait()

    plsc.subcore_barrier()

    off = sid * PER_TILE
    for c in range(2):
        @pl.when(cid == c)
        def _dump():
            pltpu.sync_copy(counts_sp.at[pl.ds(off, PER_TILE)],
                            parts[c].at[pl.ds(off, PER_TILE)])

            @pl.when(sid == 15)
            def _dump_tail():
                pltpu.sync_copy(counts_sp.at[pl.ds(TAIL_OFF, TAIL)],
                                parts[c].at[pl.ds(TAIL_OFF, TAIL)])


_sc_kernel = functools.partial(
    pl.kernel,
    out_type=[
        jax.ShapeDtypeStruct((TOTAL, EMBED_DIM), jnp.float32),
        jax.ShapeDtypeStruct((TOTAL,), jnp.int32),
        jax.ShapeDtypeStruct((ZCH,), jnp.float32),
        jax.ShapeDtypeStruct((ZCH,), jnp.float32),
    ],
    mesh=plsc.VectorSubcoreMesh(core_axis_name="c", subcore_axis_name="s"),
    compiler_params=pltpu.CompilerParams(use_tc_tiling_on_sc=False),
    scratch_types=[
        pltpu.VMEM((CHUNK,), jnp.int32),
        pltpu.VMEM((NJ, ROWS), jnp.int32),
        pltpu.VMEM((GK, ROWS, EMBED_DIM), jnp.float32),
        pltpu.VMEM((GK, ROWS, EMBED_DIM), jnp.float32),
        pltpu.VMEM((ROWS,), jnp.float32),
        pltpu.VMEM((ZCHUNK,), jnp.float32),
        pltpu.VMEM_SHARED((ZCH,), jnp.float32),
        pltpu.SemaphoreType.DMA,
        pltpu.SemaphoreType.DMA,
        pltpu.SemaphoreType.DMA,
        pltpu.SemaphoreType.DMA,
        pltpu.SemaphoreType.DMA,
    ],
)(_sc_body)


def _combine_body(p0_ref, p1_ref, o_ref):
    o_ref[...] = p0_ref[...] + p1_ref[...]


def kernel(values, lengths, table):
    del lengths
    emb, remapped, part0, part1 = _sc_kernel(values, table)
    counts = pl.pallas_call(
        _combine_body,
        out_shape=jax.ShapeDtypeStruct((ZCH,), jnp.float32),
    )(part0, part1)
    return emb, remapped, counts
